# baseline (speedup 1.0000x reference)
import functools

import jax
import jax.numpy as jnp
from jax import lax
from jax.experimental import pallas as pl
from jax.experimental.pallas import tpu as pltpu
from jax.experimental.pallas import tpu_sc as plsc

SCALE = 8.0

_NC = 2
_NS = 16
_NW = _NC * _NS
L = 16


@functools.lru_cache(maxsize=None)
def _make_widen(V, D):
    BLK = 1000

    def body(t_ref, w_ref):
        v = t_ref[...] * SCALE
        w_ref[:, 0:D] = v
        w_ref[:, D:2 * D] = v

    return pl.pallas_call(
        body,
        grid=(V // BLK,),
        in_specs=[pl.BlockSpec((BLK, D), lambda i: (i, 0))],
        out_specs=pl.BlockSpec((BLK, 2 * D), lambda i: (i, 0)),
        out_shape=jax.ShapeDtypeStruct((V, 2 * D), jnp.float32),
    )


@functools.lru_cache(maxsize=None)
def _make_gather(R, T, V, D):
    r_per_w = R // _NW
    NBUF = 2
    assert r_per_w % NBUF == 0
    n_outer = r_per_w // NBUF
    mesh = plsc.VectorSubcoreMesh(core_axis_name="c", subcore_axis_name="s")

    @functools.partial(
        pl.kernel,
        mesh=mesh,
        compiler_params=pltpu.CompilerParams(use_tc_tiling_on_sc=True),
        out_type=jax.ShapeDtypeStruct((R, T, D), jnp.float32),
        scratch_types=[
            tuple(pltpu.VMEM((T,), jnp.int32) for _ in range(NBUF)),
            tuple(pltpu.VMEM((T, 2 * D), jnp.float32) for _ in range(NBUF)),
            tuple(pltpu.VMEM((T, D), jnp.float32) for _ in range(NBUF)),
            tuple(pltpu.SemaphoreType.DMA for _ in range(NBUF)),
            tuple(pltpu.SemaphoreType.DMA for _ in range(NBUF)),
            tuple(pltpu.SemaphoreType.DMA for _ in range(NBUF)),
        ],
    )
    def gather(xf_hbm, wide_hbm, out_hbm, idx_bufs, gbufs, obufs,
               isems, gsems, ssems):
        wid = lax.axis_index("s") * _NC + lax.axis_index("c")
        xr0 = wid * r_per_w

        def start_chunk(c, b):
            pltpu.make_async_copy(
                xf_hbm.at[pl.ds((xr0 + c) * T, T)], idx_bufs[b], isems[b]
            ).start()
            pltpu.make_async_copy(
                xf_hbm.at[pl.ds((xr0 + c) * T, T)], idx_bufs[b], isems[b]
            ).wait()
            pltpu.make_async_copy(
                wide_hbm.at[idx_bufs[b]], gbufs[b], gsems[b]
            ).start()

        for b in range(NBUF):
            start_chunk(b, b)

        def outer(i, carry):
            for b in range(NBUF):
                c = i * NBUF + b
                pltpu.make_async_copy(
                    wide_hbm.at[idx_bufs[b]], gbufs[b], gsems[b]
                ).wait()

                def repack(t, carry2):
                    for j in range(D // L):
                        obufs[b][t, pl.ds(j * L, L)] = (
                            gbufs[b][t, pl.ds(j * L, L)]
                        )
                    return carry2

                lax.fori_loop(0, T, repack, 0, unroll=4)

                store = pltpu.make_async_copy(
                    obufs[b], out_hbm.at[xr0 + c], ssems[b]
                )
                store.start()

                @pl.when(c + NBUF < r_per_w)
                def _():
                    store.wait()
                    start_chunk(c + NBUF, b)

            return carry

        lax.fori_loop(0, n_outer, outer, 0)

        for b in range(NBUF):
            pltpu.make_async_copy(
                obufs[b], out_hbm.at[xr0], ssems[b]
            ).wait()

    return gather


def kernel(x, table):
    R, T = x.shape
    V, D = table.shape
    xf = x.reshape(R * T)
    wide = _make_widen(V, D)(table)
    return _make_gather(R, T, V, D)(xf, wide)


# speedup vs baseline: 1.8764x; 1.8764x over previous
import functools

import jax
import jax.numpy as jnp
from jax import lax
from jax.experimental import pallas as pl
from jax.experimental.pallas import tpu as pltpu
from jax.experimental.pallas import tpu_sc as plsc

SCALE = 8.0

_NC = 2
_NS = 16
_NW = _NC * _NS
L = 16


@functools.lru_cache(maxsize=None)
def _make_widen(V, D):
    BLK = 4096

    def body(tT_ref, w_ref):
        v = jnp.transpose(tT_ref[...]) * SCALE
        w_ref[:, 0:D] = v
        w_ref[:, D:2 * D] = v

    return pl.pallas_call(
        body,
        grid=(pl.cdiv(V, BLK),),
        in_specs=[pl.BlockSpec((D, BLK), lambda i: (0, i))],
        out_specs=pl.BlockSpec((BLK, 2 * D), lambda i: (i, 0)),
        out_shape=jax.ShapeDtypeStruct((V, 2 * D), jnp.float32),
    )


@functools.lru_cache(maxsize=None)
def _make_gather(R, T, V, D):
    r_per_w = R // _NW
    NBUF = 2
    NIB = 4
    assert r_per_w % (2 * NIB) == 0
    n_outer = r_per_w // NIB
    mesh = plsc.VectorSubcoreMesh(core_axis_name="c", subcore_axis_name="s")

    @functools.partial(
        pl.kernel,
        mesh=mesh,
        compiler_params=pltpu.CompilerParams(use_tc_tiling_on_sc=True),
        out_type=jax.ShapeDtypeStruct((R, T, D), jnp.float32),
        scratch_types=[
            tuple(pltpu.VMEM((T,), jnp.int32) for _ in range(NIB)),
            tuple(pltpu.VMEM((T, 2 * D), jnp.float32) for _ in range(NBUF)),
            tuple(pltpu.VMEM((T, D), jnp.float32) for _ in range(NBUF)),
            tuple(pltpu.SemaphoreType.DMA for _ in range(NIB)),
            tuple(pltpu.SemaphoreType.DMA for _ in range(NBUF)),
            tuple(pltpu.SemaphoreType.DMA for _ in range(NBUF)),
        ],
    )
    def gather(xf_hbm, wide_hbm, out_hbm, idx_bufs, gbufs, obufs,
               isems, gsems, ssems):
        wid = lax.axis_index("s") * _NC + lax.axis_index("c")
        xr0 = wid * r_per_w

        def idx_start(c, q):
            pltpu.make_async_copy(
                xf_hbm.at[pl.ds((xr0 + c) * T, T)], idx_bufs[q], isems[q]
            ).start()

        def idx_wait(q):
            pltpu.make_async_copy(
                xf_hbm.at[pl.ds(xr0 * T, T)], idx_bufs[q], isems[q]
            ).wait()

        def gather_start(q, b):
            pltpu.make_async_copy(
                wide_hbm.at[idx_bufs[q]], gbufs[b], gsems[b]
            ).start()

        def gather_wait(q, b):
            pltpu.make_async_copy(
                wide_hbm.at[idx_bufs[q]], gbufs[b], gsems[b]
            ).wait()

        for q in range(NIB - 1):
            idx_start(q, q)
        for c in range(NBUF):
            idx_wait(c)
            gather_start(c, c)

        def outer(i, carry):
            for k in range(NIB):
                c = i * NIB + k
                q = k
                b = k % NBUF

                @pl.when(c + NIB - 1 < r_per_w)
                def _():
                    idx_start(c + NIB - 1, (q + NIB - 1) % NIB)

                gather_wait(q, b)

                @pl.when(c >= NBUF)
                def _():
                    pltpu.make_async_copy(
                        obufs[b], out_hbm.at[xr0], ssems[b]
                    ).wait()

                def repack(t, carry2):
                    for j in range(D // L):
                        obufs[b][t, pl.ds(j * L, L)] = (
                            gbufs[b][t, pl.ds(j * L, L)]
                        )
                    return carry2

                lax.fori_loop(0, T, repack, 0, unroll=4)

                pltpu.make_async_copy(
                    obufs[b], out_hbm.at[xr0 + c], ssems[b]
                ).start()

                @pl.when(c + NBUF < r_per_w)
                def _():
                    idx_wait((q + NBUF) % NIB)
                    gather_start((q + NBUF) % NIB, b)

            return carry

        lax.fori_loop(0, n_outer, outer, 0)

        for b in range(NBUF):
            pltpu.make_async_copy(
                obufs[b], out_hbm.at[xr0], ssems[b]
            ).wait()

    return gather


def kernel(x, table):
    R, T = x.shape
    V, D = table.shape
    xf = x.reshape(R * T)
    wide = _make_widen(V, D)(table.T)
    return _make_gather(R, T, V, D)(xf, wide)


# speedup vs baseline: 1.9728x; 1.0514x over previous
import functools

import jax
import jax.numpy as jnp
from jax import lax
from jax.experimental import pallas as pl
from jax.experimental.pallas import tpu as pltpu
from jax.experimental.pallas import tpu_sc as plsc

SCALE = 8.0

_NC = 2
_NS = 16
_NW = _NC * _NS
L = 16


@functools.lru_cache(maxsize=None)
def _make_widen(V, D):
    BLK = 4096

    def body(tT_ref, w_ref):
        v = jnp.transpose(tT_ref[...]) * SCALE
        w_ref[:, 0:D] = v

    return pl.pallas_call(
        body,
        grid=(pl.cdiv(V, BLK),),
        in_specs=[pl.BlockSpec((D, BLK), lambda i: (0, i))],
        out_specs=pl.BlockSpec((BLK, 2 * D), lambda i: (i, 0)),
        out_shape=jax.ShapeDtypeStruct((V, 2 * D), jnp.float32),
    )


@functools.lru_cache(maxsize=None)
def _make_gather(R, T, V, D):
    r_per_w = R // _NW
    NBUF = 2
    NIB = 4
    assert r_per_w % (2 * NIB) == 0
    n_outer = r_per_w // NIB
    mesh = plsc.VectorSubcoreMesh(core_axis_name="c", subcore_axis_name="s")

    @functools.partial(
        pl.kernel,
        mesh=mesh,
        compiler_params=pltpu.CompilerParams(use_tc_tiling_on_sc=True),
        out_type=jax.ShapeDtypeStruct((R, T, D), jnp.float32),
        scratch_types=[
            tuple(pltpu.VMEM((T,), jnp.int32) for _ in range(NIB)),
            tuple(pltpu.VMEM((T, 2 * D), jnp.float32) for _ in range(NBUF)),
            tuple(pltpu.VMEM((T, D), jnp.float32) for _ in range(NBUF)),
            tuple(pltpu.SemaphoreType.DMA for _ in range(NIB)),
            tuple(pltpu.SemaphoreType.DMA for _ in range(NBUF)),
            tuple(pltpu.SemaphoreType.DMA for _ in range(NBUF)),
        ],
    )
    def gather(xf_hbm, wide_hbm, out_hbm, idx_bufs, gbufs, obufs,
               isems, gsems, ssems):
        wid = lax.axis_index("s") * _NC + lax.axis_index("c")
        xr0 = wid * r_per_w

        def idx_start(c, q):
            pltpu.make_async_copy(
                xf_hbm.at[pl.ds((xr0 + c) * T, T)], idx_bufs[q], isems[q]
            ).start()

        def idx_wait(q):
            pltpu.make_async_copy(
                xf_hbm.at[pl.ds(xr0 * T, T)], idx_bufs[q], isems[q]
            ).wait()

        def gather_start(q, b):
            pltpu.make_async_copy(
                wide_hbm.at[idx_bufs[q]], gbufs[b], gsems[b]
            ).start()

        def gather_wait(q, b):
            pltpu.make_async_copy(
                wide_hbm.at[idx_bufs[q]], gbufs[b], gsems[b]
            ).wait()

        for q in range(NIB - 1):
            idx_start(q, q)
        for c in range(NBUF):
            idx_wait(c)
            gather_start(c, c)

        def outer(i, carry):
            for k in range(NIB):
                c = i * NIB + k
                q = k
                b = k % NBUF

                @pl.when(c + NIB - 1 < r_per_w)
                def _():
                    idx_start(c + NIB - 1, (q + NIB - 1) % NIB)

                gather_wait(q, b)

                @pl.when(c >= NBUF)
                def _():
                    pltpu.make_async_copy(
                        obufs[b], out_hbm.at[xr0], ssems[b]
                    ).wait()

                def repack(t, carry2):
                    for j in range(D // L):
                        obufs[b][t, pl.ds(j * L, L)] = (
                            gbufs[b][t, pl.ds(j * L, L)]
                        )
                    return carry2

                lax.fori_loop(0, T, repack, 0, unroll=4)

                pltpu.make_async_copy(
                    obufs[b], out_hbm.at[xr0 + c], ssems[b]
                ).start()

                @pl.when(c + NBUF < r_per_w)
                def _():
                    idx_wait((q + NBUF) % NIB)
                    gather_start((q + NBUF) % NIB, b)

            return carry

        lax.fori_loop(0, n_outer, outer, 0)

        for b in range(NBUF):
            pltpu.make_async_copy(
                obufs[b], out_hbm.at[xr0], ssems[b]
            ).wait()

    return gather


def kernel(x, table):
    R, T = x.shape
    V, D = table.shape
    xf = x.reshape(R * T)
    wide = _make_widen(V, D)(table.T)
    return _make_gather(R, T, V, D)(xf, wide)


# speedup vs baseline: 2.3865x; 1.2097x over previous
import functools

import jax
import jax.numpy as jnp
from jax import lax
from jax.experimental import pallas as pl
from jax.experimental.pallas import tpu as pltpu
from jax.experimental.pallas import tpu_sc as plsc

SCALE = 8.0

_NC = 2
_NS = 16
_NW = _NC * _NS
L = 16


@functools.lru_cache(maxsize=None)
def _make_widen(V, D):
    BLK = 4096

    def body(tT_ref, w_ref):
        v = jnp.transpose(tT_ref[...]) * SCALE
        w_ref[:, 0:D] = v

    return pl.pallas_call(
        body,
        grid=(pl.cdiv(V, BLK),),
        in_specs=[pl.BlockSpec((D, BLK), lambda i: (0, i))],
        out_specs=pl.BlockSpec((BLK, 2 * D), lambda i: (i, 0)),
        out_shape=jax.ShapeDtypeStruct((V, 2 * D), jnp.float32),
    )


@functools.lru_cache(maxsize=None)
def _make_gather(R, T, V, D):
    r_per_w = R // _NW
    NBUF = 4
    NIB = 8
    assert r_per_w % NIB == 0
    n_outer = r_per_w // NIB
    mesh = plsc.VectorSubcoreMesh(core_axis_name="c", subcore_axis_name="s")

    @functools.partial(
        pl.kernel,
        mesh=mesh,
        compiler_params=pltpu.CompilerParams(use_tc_tiling_on_sc=True),
        out_type=jax.ShapeDtypeStruct((R, T, 2 * D), jnp.float32),
        scratch_types=[
            tuple(pltpu.VMEM((T,), jnp.int32) for _ in range(NIB)),
            tuple(pltpu.VMEM((T, 2 * D), jnp.float32) for _ in range(NBUF)),
            tuple(pltpu.SemaphoreType.DMA for _ in range(NIB)),
            tuple(pltpu.SemaphoreType.DMA for _ in range(NBUF)),
            tuple(pltpu.SemaphoreType.DMA for _ in range(NBUF)),
        ],
    )
    def gather(xf_hbm, wide_hbm, out_hbm, idx_bufs, gbufs,
               isems, gsems, ssems):
        wid = lax.axis_index("s") * _NC + lax.axis_index("c")
        xr0 = wid * r_per_w

        def idx_start(c, q):
            pltpu.make_async_copy(
                xf_hbm.at[pl.ds((xr0 + c) * T, T)], idx_bufs[q], isems[q]
            ).start()

        def idx_wait(q):
            pltpu.make_async_copy(
                xf_hbm.at[pl.ds(xr0 * T, T)], idx_bufs[q], isems[q]
            ).wait()

        def gather_start(q, b):
            pltpu.make_async_copy(
                wide_hbm.at[idx_bufs[q]], gbufs[b], gsems[b]
            ).start()

        def gather_wait(q, b):
            pltpu.make_async_copy(
                wide_hbm.at[idx_bufs[q]], gbufs[b], gsems[b]
            ).wait()

        def store_start(c, b):
            pltpu.make_async_copy(
                gbufs[b], out_hbm.at[xr0 + c], ssems[b]
            ).start()

        def store_wait(b):
            pltpu.make_async_copy(
                gbufs[b], out_hbm.at[xr0], ssems[b]
            ).wait()

        for q in range(NIB):
            idx_start(q, q)
        for c in range(NBUF):
            idx_wait(c)
            gather_start(c, c)

        def outer(i, carry):
            for k in range(NIB):
                c = i * NIB + k
                q = k
                b = k % NBUF

                gather_wait(q, b)
                store_start(c, b)

                @pl.when(c + NIB < r_per_w)
                def _():
                    idx_start(c + NIB, q)

                @pl.when(c + NBUF < r_per_w)
                def _():
                    store_wait(b)
                    idx_wait((q + NBUF) % NIB)
                    gather_start((q + NBUF) % NIB, b)

            return carry

        lax.fori_loop(0, n_outer, outer, 0)

        for b in range(NBUF):
            store_wait(b)

    return gather


def kernel(x, table):
    R, T = x.shape
    V, D = table.shape
    xf = x.reshape(R * T)
    wide = _make_widen(V, D)(table.T)
    out128 = _make_gather(R, T, V, D)(xf, wide)
    return out128[:, :, :D]


# speedup vs baseline: 2.6171x; 1.0966x over previous
import functools

import jax
import jax.numpy as jnp
from jax import lax
from jax.experimental import pallas as pl
from jax.experimental.pallas import tpu as pltpu
from jax.experimental.pallas import tpu_sc as plsc

SCALE = 8.0

_NC = 2
_NS = 16
_NW = _NC * _NS
L = 16


@functools.lru_cache(maxsize=None)
def _make_widen(V, D):
    BLK = 8192

    def body(tT_ref, w_ref):
        v = jnp.transpose(tT_ref[...]) * SCALE
        w_ref[:, 0:D] = v

    return pl.pallas_call(
        body,
        grid=(pl.cdiv(V, BLK),),
        in_specs=[pl.BlockSpec((D, BLK), lambda i: (0, i))],
        out_specs=pl.BlockSpec((BLK, 2 * D), lambda i: (i, 0)),
        out_shape=jax.ShapeDtypeStruct((V, 2 * D), jnp.float32),
    )


@functools.lru_cache(maxsize=None)
def _make_gather(R, T, V, D):
    r_per_w = R // _NW
    NBUF = 4
    NIB = 8
    assert r_per_w % NIB == 0
    n_outer = r_per_w // NIB
    mesh = plsc.VectorSubcoreMesh(core_axis_name="c", subcore_axis_name="s")

    @functools.partial(
        pl.kernel,
        mesh=mesh,
        compiler_params=pltpu.CompilerParams(use_tc_tiling_on_sc=True),
        out_type=jax.ShapeDtypeStruct((R, T, 2 * D), jnp.float32),
        scratch_types=[
            tuple(pltpu.VMEM((T,), jnp.int32) for _ in range(NIB)),
            tuple(pltpu.VMEM((T, 2 * D), jnp.float32) for _ in range(NBUF)),
            tuple(pltpu.SemaphoreType.DMA for _ in range(NIB)),
            tuple(pltpu.SemaphoreType.DMA for _ in range(NBUF)),
            tuple(pltpu.SemaphoreType.DMA for _ in range(NBUF)),
        ],
    )
    def gather(xf_hbm, wide_hbm, out_hbm, idx_bufs, gbufs,
               isems, gsems, ssems):
        wid = lax.axis_index("s") * _NC + lax.axis_index("c")
        xr0 = wid * r_per_w

        def idx_start(c, q):
            pltpu.make_async_copy(
                xf_hbm.at[pl.ds((xr0 + c) * T, T)], idx_bufs[q], isems[q]
            ).start()

        def idx_wait(q):
            pltpu.make_async_copy(
                xf_hbm.at[pl.ds(xr0 * T, T)], idx_bufs[q], isems[q]
            ).wait()

        def gather_start(q, b):
            pltpu.make_async_copy(
                wide_hbm.at[idx_bufs[q]], gbufs[b], gsems[b]
            ).start()

        def gather_wait(q, b):
            pltpu.make_async_copy(
                wide_hbm.at[idx_bufs[q]], gbufs[b], gsems[b]
            ).wait()

        def store_start(c, b):
            pltpu.make_async_copy(
                gbufs[b], out_hbm.at[xr0 + c], ssems[b]
            ).start()

        def store_wait(b):
            pltpu.make_async_copy(
                gbufs[b], out_hbm.at[xr0], ssems[b]
            ).wait()

        for q in range(NIB):
            idx_start(q, q)
        for c in range(NBUF):
            idx_wait(c)
            gather_start(c, c)

        def outer(i, carry):
            for k in range(NIB):
                c = i * NIB + k
                q = k
                b = k % NBUF

                gather_wait(q, b)
                store_start(c, b)

                @pl.when(c + NIB < r_per_w)
                def _():
                    idx_start(c + NIB, q)

                @pl.when(c + NBUF < r_per_w)
                def _():
                    store_wait(b)
                    idx_wait((q + NBUF) % NIB)
                    gather_start((q + NBUF) % NIB, b)

            return carry

        lax.fori_loop(0, n_outer, outer, 0)

        for b in range(NBUF):
            store_wait(b)

    return gather


def kernel(x, table):
    R, T = x.shape
    V, D = table.shape
    xf = x.reshape(R * T)
    wide = _make_widen(V, D)(table.T)
    out128 = _make_gather(R, T, V, D)(xf, wide)
    return out128[:, :, :D]


# speedup vs baseline: 2.6913x; 1.0283x over previous
import functools

import jax
import jax.numpy as jnp
from jax import lax
from jax.experimental import pallas as pl
from jax.experimental.pallas import tpu as pltpu
from jax.experimental.pallas import tpu_sc as plsc

SCALE = 8.0

_NC = 2
_NS = 16
_NW = _NC * _NS
L = 16


@functools.lru_cache(maxsize=None)
def _make_widen(V, D):
    BLK = 16384

    def body(tT_ref, w_ref):
        v = jnp.transpose(tT_ref[...]) * SCALE
        w_ref[:, 0:D] = v

    return pl.pallas_call(
        body,
        grid=(pl.cdiv(V, BLK),),
        in_specs=[pl.BlockSpec((D, BLK), lambda i: (0, i))],
        out_specs=pl.BlockSpec((BLK, 2 * D), lambda i: (i, 0)),
        out_shape=jax.ShapeDtypeStruct((V, 2 * D), jnp.float32),
    )


@functools.lru_cache(maxsize=None)
def _make_gather(R, T, V, D):
    r_per_w = R // _NW
    NBUF = 4
    NIB = 8
    assert r_per_w % NIB == 0
    n_outer = r_per_w // NIB
    mesh = plsc.VectorSubcoreMesh(core_axis_name="c", subcore_axis_name="s")

    @functools.partial(
        pl.kernel,
        mesh=mesh,
        compiler_params=pltpu.CompilerParams(use_tc_tiling_on_sc=True),
        out_type=jax.ShapeDtypeStruct((R, T, 2 * D), jnp.float32),
        scratch_types=[
            tuple(pltpu.VMEM((T,), jnp.int32) for _ in range(NIB)),
            tuple(pltpu.VMEM((T, 2 * D), jnp.float32) for _ in range(NBUF)),
            tuple(pltpu.SemaphoreType.DMA for _ in range(NIB)),
            tuple(pltpu.SemaphoreType.DMA for _ in range(NBUF)),
            tuple(pltpu.SemaphoreType.DMA for _ in range(NBUF)),
        ],
    )
    def gather(xf_hbm, wide_hbm, out_hbm, idx_bufs, gbufs,
               isems, gsems, ssems):
        wid = lax.axis_index("s") * _NC + lax.axis_index("c")
        xr0 = wid * r_per_w

        def idx_start(c, q):
            pltpu.make_async_copy(
                xf_hbm.at[pl.ds((xr0 + c) * T, T)], idx_bufs[q], isems[q]
            ).start()

        def idx_wait(q):
            pltpu.make_async_copy(
                xf_hbm.at[pl.ds(xr0 * T, T)], idx_bufs[q], isems[q]
            ).wait()

        def gather_start(q, b):
            pltpu.make_async_copy(
                wide_hbm.at[idx_bufs[q]], gbufs[b], gsems[b]
            ).start()

        def gather_wait(q, b):
            pltpu.make_async_copy(
                wide_hbm.at[idx_bufs[q]], gbufs[b], gsems[b]
            ).wait()

        def store_start(c, b):
            pltpu.make_async_copy(
                gbufs[b], out_hbm.at[xr0 + c], ssems[b]
            ).start()

        def store_wait(b):
            pltpu.make_async_copy(
                gbufs[b], out_hbm.at[xr0], ssems[b]
            ).wait()

        for q in range(NIB):
            idx_start(q, q)
        for c in range(NBUF):
            idx_wait(c)
            gather_start(c, c)

        def outer(i, carry):
            for k in range(NIB):
                c = i * NIB + k
                q = k
                b = k % NBUF

                gather_wait(q, b)
                store_start(c, b)

                @pl.when(c + NIB < r_per_w)
                def _():
                    idx_start(c + NIB, q)

                @pl.when(c + NBUF < r_per_w)
                def _():
                    store_wait(b)
                    idx_wait((q + NBUF) % NIB)
                    gather_start((q + NBUF) % NIB, b)

            return carry

        lax.fori_loop(0, n_outer, outer, 0)

        for b in range(NBUF):
            store_wait(b)

    return gather


def kernel(x, table):
    R, T = x.shape
    V, D = table.shape
    xf = x.reshape(R * T)
    wide = _make_widen(V, D)(table.T)
    out128 = _make_gather(R, T, V, D)(xf, wide)
    return out128[:, :, :D]
